# Initial kernel scaffold; baseline (speedup 1.0000x reference)
import jax
import jax.numpy as jnp
from jax.experimental import pallas as pl


def kernel(feats, graph, W_conv, b_conv, W_gate, w1, b1, w2, b2, gamma, beta):
    raise NotImplementedError("write your pallas kernel here")


# speedup vs baseline: 6.8659x; 6.8659x over previous
import functools

import jax
import jax.numpy as jnp
from jax import lax
from jax.experimental import pallas as pl
from jax.experimental.pallas import tpu as pltpu
from jax.experimental.pallas import tpu_sc as plsc

N_NODES = 10000
N_EDGES = 320000
D_MODEL = 128
D_FF = 256
N_EXP = 64

N_PAD = 10240
ROWS_PER_TILE = 640
EDGE_BLK = 80
EDGE_NBLK = N_EDGES // EDGE_BLK
BLK_PER_WORKER = EDGE_NBLK // 32

_mesh = plsc.VectorSubcoreMesh(core_axis_name="c", subcore_axis_name="s")


@functools.partial(
    pl.kernel,
    out_type=[
        jax.ShapeDtypeStruct((2, N_PAD, D_MODEL), jnp.float32),
        jax.ShapeDtypeStruct((2, N_PAD), jnp.float32),
    ],
    mesh=_mesh,
    scratch_types=[
        pltpu.VMEM_SHARED((N_PAD, D_MODEL), jnp.float32),
        pltpu.VMEM_SHARED((N_PAD,), jnp.float32),
        pltpu.VMEM((BLK_PER_WORKER, EDGE_BLK), jnp.int32),
        pltpu.VMEM((BLK_PER_WORKER, EDGE_BLK), jnp.int32),
        pltpu.VMEM((EDGE_BLK, D_MODEL), jnp.float32),
        pltpu.VMEM((8, D_MODEL), jnp.float32),
        pltpu.VMEM((EDGE_BLK,), jnp.float32),
        pltpu.SemaphoreType.DMA,
    ],
)
def _sc_agg(feats_hbm, src_hbm, dst_hbm, parts_hbm, degp_hbm,
            agg_sh, deg_sh, src_v, dst_v, rows_v, zbuf, ones_v, sem):
    c = lax.axis_index("c")
    s = lax.axis_index("s")
    wid = c * 16 + s
    row0 = s * ROWS_PER_TILE

    @pl.loop(0, 8)
    def _(r):
        for k8 in range(D_MODEL // 16):
            zbuf[r, pl.ds(k8 * 16, 16)] = jnp.zeros((16,), jnp.float32)

    @pl.loop(0, EDGE_BLK // 16)
    def _(r):
        ones_v[pl.ds(r * 16, 16)] = jnp.ones((16,), jnp.float32)

    @pl.loop(0, ROWS_PER_TILE // 8)
    def _(jz):
        pltpu.sync_copy(zbuf, agg_sh.at[pl.ds(row0 + jz * 8, 8)])

    @pl.loop(0, ROWS_PER_TILE // D_MODEL)
    def _(jz):
        pltpu.sync_copy(zbuf.at[0], deg_sh.at[pl.ds(row0 + jz * D_MODEL, D_MODEL)])

    plsc.subcore_barrier()

    pltpu.sync_copy(src_hbm.at[wid], src_v)
    pltpu.sync_copy(dst_hbm.at[wid], dst_v)

    @pl.loop(0, BLK_PER_WORKER)
    def _(j):
        pltpu.async_copy(feats_hbm.at[src_v.at[j]], rows_v, sem).wait()
        pltpu.sync_copy(rows_v, agg_sh.at[dst_v.at[j]], add=True)
        pltpu.sync_copy(ones_v, deg_sh.at[dst_v.at[j]], add=True)

    plsc.subcore_barrier()

    pltpu.sync_copy(agg_sh.at[pl.ds(row0, ROWS_PER_TILE)],
                    parts_hbm.at[c, pl.ds(row0, ROWS_PER_TILE)])
    pltpu.sync_copy(deg_sh.at[pl.ds(row0, ROWS_PER_TILE)],
                    degp_hbm.at[c, pl.ds(row0, ROWS_PER_TILE)])


_B = 2048
_NB = N_PAD // _B


def _main_body(parts_ref, invd_ref, wc_ref, bc_ref, wg_ref,
               w1_ref, b1_ref, w2_ref, b2_ref,
               x_ref, bn_ref, h_s, acc_s, g1_s, g2_s, i1_s, i2_s):
    i = pl.program_id(0)
    e = pl.program_id(1)

    @pl.when(e == 0)
    def _():
        agg = parts_ref[0] + parts_ref[1]
        h = agg * invd_ref[...][:, None]
        h = jnp.dot(h, wc_ref[...], preferred_element_type=jnp.float32)
        h = h + bc_ref[...][None, :]
        h_s[...] = h
        logits = jnp.dot(h, wg_ref[...], preferred_element_type=jnp.float32)
        iota = lax.broadcasted_iota(jnp.int32, (_B, N_EXP), 1)
        m1 = jnp.max(logits, axis=-1, keepdims=True)
        i1 = jnp.min(jnp.where(logits == m1, iota, N_EXP), axis=-1, keepdims=True)
        l2 = jnp.where(iota == i1, -jnp.inf, logits)
        m2 = jnp.max(l2, axis=-1, keepdims=True)
        i2 = jnp.min(jnp.where(l2 == m2, iota, N_EXP), axis=-1, keepdims=True)
        r = jnp.exp(m2 - m1)
        g1_s[...] = 1.0 / (1.0 + r)
        g2_s[...] = r / (1.0 + r)
        i1_s[...] = i1
        i2_s[...] = i2
        acc_s[...] = jnp.zeros((_B, D_MODEL), jnp.float32)

    h = h_s[...]
    hid = jnp.dot(h, w1_ref[0], preferred_element_type=jnp.float32)
    hid = hid + b1_ref[0]
    hid = 0.5 * hid * (1.0 + lax.erf(hid * 0.7071067811865476))
    yo = jnp.dot(hid, w2_ref[0], preferred_element_type=jnp.float32)
    yo = yo + b2_ref[0]
    ge = (jnp.where(i1_s[...] == e, g1_s[...], 0.0)
          + jnp.where(i2_s[...] == e, g2_s[...], 0.0))
    acc_s[...] = acc_s[...] + yo * ge

    @pl.when(e == N_EXP - 1)
    def _():
        x = h_s[...] + acc_s[...]
        x_ref[...] = x
        row = lax.broadcasted_iota(jnp.int32, (_B, 1), 0) + i * _B
        xm = jnp.where(row < N_NODES, x, 0.0)
        s1 = jnp.sum(xm, axis=0)[None, :]
        s2 = jnp.sum(xm * xm, axis=0)[None, :]
        prev = jnp.where(i == 0, jnp.zeros((2, D_MODEL), jnp.float32), bn_ref[...])
        bn_ref[...] = prev + jnp.concatenate([s1, s2], axis=0)


def _tc_main(parts, inv_deg, W_conv, b_conv, W_gate, w1, b1, w2, b2):
    return pl.pallas_call(
        _main_body,
        grid=(_NB, N_EXP),
        in_specs=[
            pl.BlockSpec((2, _B, D_MODEL), lambda i, e: (0, i, 0)),
            pl.BlockSpec((_B,), lambda i, e: (i,)),
            pl.BlockSpec((D_MODEL, D_MODEL), lambda i, e: (0, 0)),
            pl.BlockSpec((D_MODEL,), lambda i, e: (0,)),
            pl.BlockSpec((D_MODEL, N_EXP), lambda i, e: (0, 0)),
            pl.BlockSpec((1, D_MODEL, D_FF), lambda i, e: (e, 0, 0)),
            pl.BlockSpec((1, 1, D_FF), lambda i, e: (e, 0, 0)),
            pl.BlockSpec((1, D_FF, D_MODEL), lambda i, e: (e, 0, 0)),
            pl.BlockSpec((1, 1, D_MODEL), lambda i, e: (e, 0, 0)),
        ],
        out_specs=[
            pl.BlockSpec((_B, D_MODEL), lambda i, e: (i, 0)),
            pl.BlockSpec((2, D_MODEL), lambda i, e: (0, 0)),
        ],
        out_shape=[
            jax.ShapeDtypeStruct((N_NODES, D_MODEL), jnp.float32),
            jax.ShapeDtypeStruct((2, D_MODEL), jnp.float32),
        ],
        scratch_shapes=[
            pltpu.VMEM((_B, D_MODEL), jnp.float32),
            pltpu.VMEM((_B, D_MODEL), jnp.float32),
            pltpu.VMEM((_B, 1), jnp.float32),
            pltpu.VMEM((_B, 1), jnp.float32),
            pltpu.VMEM((_B, 1), jnp.int32),
            pltpu.VMEM((_B, 1), jnp.int32),
        ],
        compiler_params=pltpu.CompilerParams(
            dimension_semantics=("arbitrary", "arbitrary")),
    )(parts, inv_deg, W_conv, b_conv, W_gate, w1,
      b1.reshape(N_EXP, 1, D_FF), w2, b2.reshape(N_EXP, 1, D_MODEL))


def _norm_body(x_ref, sc_ref, sh_ref, y_ref):
    y_ref[...] = x_ref[...] * sc_ref[...][None, :] + sh_ref[...][None, :]


def _tc_norm(x, scale, shift):
    return pl.pallas_call(
        _norm_body,
        grid=(_NB,),
        in_specs=[
            pl.BlockSpec((_B, D_MODEL), lambda i: (i, 0)),
            pl.BlockSpec((D_MODEL,), lambda i: (0,)),
            pl.BlockSpec((D_MODEL,), lambda i: (0,)),
        ],
        out_specs=pl.BlockSpec((_B, D_MODEL), lambda i: (i, 0)),
        out_shape=jax.ShapeDtypeStruct((N_NODES, D_MODEL), jnp.float32),
    )(x, scale, shift)


def kernel(feats, graph, W_conv, b_conv, W_gate, w1, b1, w2, b2, gamma, beta):
    src = graph[0].reshape(32, BLK_PER_WORKER, EDGE_BLK)
    dst = graph[1].reshape(32, BLK_PER_WORKER, EDGE_BLK)
    parts, degp = _sc_agg(feats, src, dst)
    inv_deg = 1.0 / jnp.maximum(degp[0] + degp[1], 1.0)
    x, bn = _tc_main(parts, inv_deg, W_conv, b_conv, W_gate, w1, b1, w2, b2)
    n = jnp.float32(N_NODES)
    mean = bn[0] / n
    var = jnp.maximum(bn[1] / n - mean * mean, 0.0)
    scale = gamma * lax.rsqrt(var + 1e-5)
    shift = beta - mean * scale
    return _tc_norm(x, scale, shift)
